# baseline (speedup 1.0000x reference)
import functools

import jax
import jax.numpy as jnp
from jax import lax
from jax.experimental import pallas as pl
from jax.experimental.pallas import tpu as pltpu
from jax.experimental.pallas import tpu_sc as plsc

_NC = 2
_NS = 16
_NW = _NC * _NS


@functools.lru_cache(maxsize=None)
def _make_circulant_kernel(H, S):
  n_blocks = S // 128
  n_ug = 128 // 8
  W = 2 * S - 128
  n_units = H // 2
  mesh = plsc.VectorSubcoreMesh(core_axis_name="c", subcore_axis_name="s")

  @functools.partial(
      pl.kernel,
      mesh=mesh,
      out_type=jax.ShapeDtypeStruct((H, S, S), jnp.float32),
      scratch_types=[pltpu.VMEM((8, W), jnp.float32)] * 3
      + [pltpu.SemaphoreType.DMA] * 3,
  )
  def k(e128_hbm, out_hbm, *scratch):
    bufs, sems = scratch[:3], scratch[3:]
    wid = lax.axis_index("s") * _NC + lax.axis_index("c")
    wu = lax.rem(wid, n_ug)
    hp = wid // n_ug
    u0 = pl.multiple_of(8 * wu, 8)

    def issue_load(t, b):
      h = hp + 2 * t
      pltpu.async_copy(e128_hbm.at[h, pl.ds(u0, 8)],
                       bufs[b].at[:, pl.ds(0, S)], sems[b])
      pltpu.async_copy(e128_hbm.at[h, pl.ds(u0, 8), pl.ds(0, W - S)],
                       bufs[b].at[:, pl.ds(S, W - S)], sems[b])

    def wait_load(b):
      pltpu.make_async_copy(e128_hbm.at[0, pl.ds(0, 8)],
                            bufs[b].at[:, pl.ds(0, S)], sems[b]).wait()
      pltpu.make_async_copy(e128_hbm.at[0, pl.ds(0, 8), pl.ds(0, W - S)],
                            bufs[b].at[:, pl.ds(S, W - S)], sems[b]).wait()

    def issue_stores(t, b):
      h = hp + 2 * t
      for kb in range(n_blocks):
        i_row = pl.multiple_of(128 * kb + u0, 8)
        pltpu.async_copy(bufs[b].at[:, pl.ds(S - 128 - 128 * kb, S)],
                         out_hbm.at[h, pl.ds(i_row, 8)], sems[b])

    def drain_stores(b):
      for kb in range(n_blocks):
        pltpu.make_async_copy(bufs[b].at[:, pl.ds(0, S)],
                              out_hbm.at[0, pl.ds(0, 8)], sems[b]).wait()

    issue_load(0, 0)
    if n_units > 1:
      issue_load(1, 1)
    for t in range(n_units):
      b = t % 3
      wait_load(b)
      issue_stores(t, b)
      if t + 2 < n_units:
        if t >= 1:
          drain_stores((t + 2) % 3)
        issue_load(t + 2, (t + 2) % 3)
    for t in range(max(0, n_units - 3), n_units):
      drain_stores(t % 3)

  return k


def _expand_body(t_ref, out_ref):
  s = t_ref.shape[-1]
  v = pltpu.roll(t_ref[0], s - 127, axis=1)
  x = jnp.broadcast_to(v, (128, s))
  out_ref[...] = pltpu.roll(x, 0, axis=1, stride=1, stride_axis=0)[None]


@functools.lru_cache(maxsize=None)
def _make_expand_kernel(H, S):
  return pl.pallas_call(
      _expand_body,
      grid=(H,),
      in_specs=[pl.BlockSpec((1, 1, S), lambda h: (h, 0, 0))],
      out_specs=pl.BlockSpec((1, 128, S), lambda h: (h, 0, 0)),
      out_shape=jax.ShapeDtypeStruct((H, 128, S), jnp.float32),
  )


def kernel(seq_len, bias_table):
  del seq_len
  S, H = bias_table.shape
  t = jnp.flip(bias_table, axis=0).T[:, None, :]
  e128 = _make_expand_kernel(H, S)(t)
  return _make_circulant_kernel(H, S)(e128)
